# Initial kernel scaffold; baseline (speedup 1.0000x reference)
import jax
import jax.numpy as jnp
from jax.experimental import pallas as pl


def kernel(x, edge_index, batch, params):
    raise NotImplementedError("write your pallas kernel here")


# speedup vs baseline: 4.6121x; 4.6121x over previous
import functools

import jax
import jax.numpy as jnp
from jax import lax
from jax.experimental import pallas as pl
from jax.experimental.pallas import tpu as pltpu
from jax.experimental.pallas import tpu_sc as plsc

N = 10000
E = 320000
B = 128

NC = 2
NS = 16
NW = NC * NS

C = 128
K = (E + NW * C - 1) // (NW * C)
E_PAD = NW * K * C

NP = 10240
ROWS_PER_TILE = NP // NS
SINK = N

RBLK = 2000
G = N // RBLK


def _make_edge_agg(d):
    mesh = plsc.VectorSubcoreMesh(core_axis_name="c", subcore_axis_name="s")

    @functools.partial(
        pl.kernel,
        out_type=jax.ShapeDtypeStruct((NC, NP, d), jnp.float32),
        mesh=mesh,
        compiler_params=pltpu.CompilerParams(use_tc_tiling_on_sc=False),
        scratch_types=[
            pltpu.VMEM((K, C), jnp.int32),
            pltpu.VMEM((K, C), jnp.int32),
            pltpu.VMEM((C, d), jnp.float32),
            pltpu.VMEM_SHARED((NP, d), jnp.float32),
            pltpu.SemaphoreType.DMA,
        ],
    )
    def edge_agg(feat_hbm, srcs_hbm, dsts_hbm, zeros_hbm, out_hbm,
                 src_v, dst_v, rows_v, acc_sh, sem):
        c = lax.axis_index("c")
        s = lax.axis_index("s")
        wid = c * NS + s
        row0 = s * ROWS_PER_TILE

        for k in range(ROWS_PER_TILE // 128):
            pltpu.sync_copy(zeros_hbm, acc_sh.at[pl.ds(row0 + k * 128, 128)])

        pltpu.sync_copy(srcs_hbm.at[wid], src_v)
        pltpu.sync_copy(dsts_hbm.at[wid], dst_v)
        plsc.subcore_barrier()

        def body(j, carry):
            pltpu.async_copy(feat_hbm.at[src_v.at[j]], rows_v, sem).wait()
            pltpu.sync_copy(rows_v, acc_sh.at[dst_v.at[j]], add=True)
            return carry

        lax.fori_loop(0, K, body, 0)
        plsc.subcore_barrier()

        pltpu.sync_copy(acc_sh.at[pl.ds(row0, ROWS_PER_TILE)],
                        out_hbm.at[c, pl.ds(row0, ROWS_PER_TILE)])

    return edge_agg


def _mlp_stats_body(feat_r, agg0_r, agg1_r, eps_r, W1_r, b1_r, W2_r, b2_r,
                    h_r, stats_r):
    i = pl.program_id(0)
    hin = feat_r[...] * (1.0 + eps_r[0]) + agg0_r[...] + agg1_r[...]
    h1 = jnp.maximum(
        jnp.dot(hin, W1_r[...], preferred_element_type=jnp.float32) + b1_r[...],
        0.0)
    h2 = jnp.dot(h1, W2_r[...], preferred_element_type=jnp.float32) + b2_r[...]
    h_r[...] = h2

    @pl.when(i == 0)
    def _():
        stats_r[...] = jnp.zeros_like(stats_r)

    stats_r[0:1, :] += jnp.sum(h2, axis=0, keepdims=True)
    stats_r[1:2, :] += jnp.sum(h2 * h2, axis=0, keepdims=True)


def _layer_mlp(feat, agg0, agg1, eps, W1, b1, W2, b2):
    din = feat.shape[1]
    dout = W1.shape[1]
    h, stats = pl.pallas_call(
        _mlp_stats_body,
        grid=(G,),
        in_specs=[
            pl.BlockSpec((RBLK, din), lambda i: (i, 0)),
            pl.BlockSpec((RBLK, din), lambda i: (i, 0)),
            pl.BlockSpec((RBLK, din), lambda i: (i, 0)),
            pl.BlockSpec(memory_space=pltpu.SMEM),
            pl.BlockSpec((din, dout), lambda i: (0, 0)),
            pl.BlockSpec((1, dout), lambda i: (0, 0)),
            pl.BlockSpec((dout, dout), lambda i: (0, 0)),
            pl.BlockSpec((1, dout), lambda i: (0, 0)),
        ],
        out_specs=[
            pl.BlockSpec((RBLK, dout), lambda i: (i, 0)),
            pl.BlockSpec((8, dout), lambda i: (0, 0)),
        ],
        out_shape=[
            jax.ShapeDtypeStruct((N, dout), jnp.float32),
            jax.ShapeDtypeStruct((8, dout), jnp.float32),
        ],
    )(feat, agg0, agg1, eps, W1, b1, W2, b2)
    return h, stats


def _bn_pool_body(h_r, stats_r, gamma_r, beta_r, Wi_r, bi_r, Wo_r, bo_r,
                  batch_r, feat_r, pooled_r, pout_r):
    i = pl.program_id(0)
    inv_n = 1.0 / N
    mean = stats_r[0:1, :] * inv_n
    ex2 = stats_r[1:2, :] * inv_n
    var = ex2 - mean * mean
    inv = lax.rsqrt(var + 1e-5)
    f = jnp.maximum((h_r[...] - mean) * inv * gamma_r[...] + beta_r[...], 0.0)
    feat_r[...] = f
    inner = jnp.maximum(
        jnp.dot(f, Wi_r[...], preferred_element_type=jnp.float32) + bi_r[...],
        0.0)
    bids = batch_r[0, 0, :]
    onehot = (lax.broadcasted_iota(jnp.int32, (B, RBLK), 0)
              == bids[None, :]).astype(jnp.float32)

    @pl.when(i == 0)
    def _():
        pooled_r[...] = jnp.zeros_like(pooled_r)

    pooled_r[...] += jnp.dot(onehot, inner, preferred_element_type=jnp.float32)

    @pl.when(i == G - 1)
    def _():
        pout_r[...] = jnp.maximum(
            jnp.dot(pooled_r[...], Wo_r[...],
                    preferred_element_type=jnp.float32) + bo_r[...],
            0.0)


def _layer_bn_pool(h, stats, gamma, beta, Wi, bi, Wo, bo, batch3d):
    dout = h.shape[1]
    feat, _, pout = pl.pallas_call(
        _bn_pool_body,
        grid=(G,),
        in_specs=[
            pl.BlockSpec((RBLK, dout), lambda i: (i, 0)),
            pl.BlockSpec((8, dout), lambda i: (0, 0)),
            pl.BlockSpec((1, dout), lambda i: (0, 0)),
            pl.BlockSpec((1, dout), lambda i: (0, 0)),
            pl.BlockSpec((dout, dout), lambda i: (0, 0)),
            pl.BlockSpec((1, dout), lambda i: (0, 0)),
            pl.BlockSpec((dout, dout), lambda i: (0, 0)),
            pl.BlockSpec((1, dout), lambda i: (0, 0)),
            pl.BlockSpec((1, 1, RBLK), lambda i: (i, 0, 0)),
        ],
        out_specs=[
            pl.BlockSpec((RBLK, dout), lambda i: (i, 0)),
            pl.BlockSpec((B, dout), lambda i: (0, 0)),
            pl.BlockSpec((B, dout), lambda i: (0, 0)),
        ],
        out_shape=[
            jax.ShapeDtypeStruct((N, dout), jnp.float32),
            jax.ShapeDtypeStruct((B, dout), jnp.float32),
            jax.ShapeDtypeStruct((B, dout), jnp.float32),
        ],
    )(h, stats, gamma, beta, Wi, bi, Wo, bo, batch3d)
    return feat, pout


def _head_body(p1_r, p2_r, p3_r, w1a_r, w1b_r, w1c_r, b1_r, W2_r, b2_r,
               W3_r, b3_r, W4_r, b4_r, out_r):
    h = (jnp.dot(p1_r[...], w1a_r[...], preferred_element_type=jnp.float32)
         + jnp.dot(p2_r[...], w1b_r[...], preferred_element_type=jnp.float32)
         + jnp.dot(p3_r[...], w1c_r[...], preferred_element_type=jnp.float32)
         + b1_r[...])
    h = jnp.maximum(h, 0.0)
    h = jnp.tanh(
        jnp.dot(h, W2_r[...], preferred_element_type=jnp.float32) + b2_r[...])
    s = jnp.maximum(
        jnp.dot(h, W3_r[...], preferred_element_type=jnp.float32) + b3_r[...],
        0.0)
    s = jnp.dot(s, W4_r[...], preferred_element_type=jnp.float32) + b4_r[...]
    out_r[...] = 1.0 / (1.0 + jnp.exp(-s))


def _head(p1, p2, p3, w1a, w1b, w1c, b1, W2, b2, W3, b3, W4, b4):
    return pl.pallas_call(
        _head_body,
        out_shape=jax.ShapeDtypeStruct((B, 1), jnp.float32),
    )(p1, p2, p3, w1a, w1b, w1c, b1, W2, b2, W3, b3, W4, b4)


@jax.jit
def kernel(x, edge_index, batch, params):
    src = edge_index[0].astype(jnp.int32)
    dst = edge_index[1].astype(jnp.int32)
    pad = E_PAD - E
    src = jnp.concatenate([src, jnp.zeros((pad,), jnp.int32)])
    dst = jnp.concatenate([dst, jnp.full((pad,), SINK, jnp.int32)])
    srcs = src.reshape(NW, K, C)
    dsts = dst.reshape(NW, K, C)
    batch3d = batch.astype(jnp.int32).reshape(G, 1, RBLK)

    feat = x
    pouts = []
    for i in range(3):
        p = params['gin'][i]
        d = feat.shape[1]
        zeros = jnp.zeros((128, d), jnp.float32)
        aggs = _make_edge_agg(d)(feat, srcs, dsts, zeros)
        agg0 = aggs[0, :N]
        agg1 = aggs[1, :N]
        eps = jnp.reshape(p['eps'], (1,))
        h, stats = _layer_mlp(feat, agg0, agg1, eps,
                              p['W1'], p['b1'].reshape(1, -1),
                              p['W2'], p['b2'].reshape(1, -1))
        pi = params['inner'][i]
        po = params['outer'][i]
        feat, pout = _layer_bn_pool(
            h, stats, p['gamma'].reshape(1, -1), p['beta'].reshape(1, -1),
            pi['W'], pi['b'].reshape(1, -1), po['W'], po['b'].reshape(1, -1),
            batch3d)
        pouts.append(pout)

    csW1 = params['cs_W1']
    w1a, w1b, w1c = csW1[:128], csW1[128:192], csW1[192:224]
    return _head(pouts[0], pouts[1], pouts[2],
                 w1a, w1b, w1c, params['cs_b1'].reshape(1, -1),
                 params['cs_W2'], params['cs_b2'].reshape(1, -1),
                 params['sc_W1'], params['sc_b1'].reshape(1, -1),
                 params['sc_W2'], params['sc_b2'].reshape(1, -1))
